# Initial kernel scaffold; baseline (speedup 1.0000x reference)
import jax
import jax.numpy as jnp
from jax.experimental import pallas as pl


def kernel(activations, rank_weights):
    raise NotImplementedError("write your pallas kernel here")


# speedup vs baseline: 8.4572x; 8.4572x over previous
import functools

import jax
import jax.numpy as jnp
from jax.experimental import pallas as pl

_TP = 256


def _body(n_pass, a_ref, w0_ref, d_ref, o_ref):
    a = a_ref[0]
    F = a.shape[0]
    iota = jax.lax.broadcasted_iota(jnp.int32, a.shape, 0)
    work = a
    acc = jnp.broadcast_to(w0_ref[...], a.shape)
    for j in range(n_pass):
        m = jnp.max(work, axis=0, keepdims=True)
        g = (a < m).astype(a.dtype)
        acc = acc + d_ref[:, j : j + 1] * g
        if j < n_pass - 1:
            kidx = jnp.min(jnp.where(work < m, F, iota), axis=0, keepdims=True)
            work = jnp.where(iota == kidx, -jnp.inf, work)
    o_ref[0] = acc


def kernel(activations, rank_weights):
    B, F, H, W = activations.shape
    n_pass = rank_weights.shape[1]
    P = H * W
    a3 = activations.reshape(B, F, P)
    w0 = rank_weights[:, :1]
    d = jnp.concatenate(
        [rank_weights[:, 1:] - rank_weights[:, :-1], -rank_weights[:, -1:]],
        axis=1,
    )
    tp = min(_TP, P)
    out = pl.pallas_call(
        functools.partial(_body, n_pass),
        grid=(B, P // tp),
        in_specs=[
            pl.BlockSpec((1, F, tp), lambda b, p: (b, 0, p)),
            pl.BlockSpec((F, 1), lambda b, p: (0, 0)),
            pl.BlockSpec((F, n_pass), lambda b, p: (0, 0)),
        ],
        out_specs=pl.BlockSpec((1, F, tp), lambda b, p: (b, 0, p)),
        out_shape=jax.ShapeDtypeStruct((B, F, P), jnp.float32),
    )(a3, w0, d)
    return out.reshape(B, F, H, W)
